# baseline (speedup 1.0000x reference)
import functools

import jax
import jax.numpy as jnp
from jax import lax
from jax.experimental import pallas as pl
from jax.experimental.pallas import tpu as pltpu
from jax.experimental.pallas import tpu_sc as plsc

N = 10000
E = 320000
D = 128
B = 8
NC = 2
NS = 16
NW = NC * NS
EPW = E // NW
CH = 40
NCH = EPW // CH
NB = 50
NBLK = NCH // NB
RPT = N // NS
PW = 136
DW = 16


def _mm_t(a, w):
    return lax.dot_general(a, w, (((1,), (1,)), ((), ())),
                           preferred_element_type=jnp.float32)


def _k1_body(x_ref, wl_ref, wr_ref, ea_ref, xl_ref, xr_ref, ea16_ref):
    xb = x_ref[...]
    xl_ref[...] = _mm_t(xb, wl_ref[...])
    xr_ref[...] = _mm_t(xb, wr_ref[...])
    eblk = ea_ref.shape[0]
    ea16_ref[...] = jnp.concatenate(
        [jnp.zeros((eblk, 8), jnp.float32), ea_ref[...],
         jnp.ones((eblk, 1), jnp.float32),
         jnp.zeros((eblk, 3), jnp.float32)], axis=1)


def _k1(x, Wl, Wr, edge_attr):
    blk = 200
    eblk = E // (N // blk)
    return pl.pallas_call(
        _k1_body,
        grid=(N // blk,),
        in_specs=[
            pl.BlockSpec((blk, D), lambda i: (i, 0)),
            pl.BlockSpec((D, D), lambda i: (0, 0)),
            pl.BlockSpec((D, D), lambda i: (0, 0)),
            pl.BlockSpec((eblk, 4), lambda i: (i, 0)),
        ],
        out_specs=[
            pl.BlockSpec((blk, D), lambda i: (i, 0)),
            pl.BlockSpec((blk, D), lambda i: (i, 0)),
            pl.BlockSpec((eblk, DW), lambda i: (i, 0)),
        ],
        out_shape=[
            jax.ShapeDtypeStruct((N, D), jnp.float32),
            jax.ShapeDtypeStruct((N, D), jnp.float32),
            jax.ShapeDtypeStruct((E, DW), jnp.float32),
        ],
    )(x, Wl, Wr, edge_attr)


def _sc_pass(xlr, idx3, dst3, ea16, w8, z144):
    mesh = plsc.VectorSubcoreMesh(core_axis_name="c", subcore_axis_name="s",
                                  num_cores=NC, num_subcores=NS)

    out_type = [jax.ShapeDtypeStruct((NC * N, PW), jnp.float32)]

    scratch = [
        pltpu.VMEM_SHARED((N, PW), jnp.float32),
        pltpu.VMEM((NB, 2 * CH), jnp.int32),
        pltpu.VMEM((NB, CH), jnp.int32),
        pltpu.VMEM((2, CH, DW), jnp.float32),
        pltpu.VMEM((2, 2 * CH, D), jnp.float32),
        pltpu.VMEM((2, CH, PW), jnp.float32),
        pltpu.VMEM((8, D), jnp.float32),
    ] + [pltpu.SemaphoreType.DMA] * 6

    def body(xlr_h, idx_h, dst_h, ea_h, w_h, z144_h, *rest):
        (acc_o, acc, idx_blk, dst_blk, ea_v, g_v, p_v, w_v,
         *sems) = rest
        sg = sems[0:2]
        sea = sems[2:4]
        ssc = sems[4:6]

        cid = lax.axis_index("c")
        sid = lax.axis_index("s")
        wid = sid * NC + cid
        base = wid * EPW
        r0 = sid * RPT

        pltpu.sync_copy(z144_h.at[pl.ds(r0, RPT)], acc.at[pl.ds(r0, RPT)])
        pltpu.sync_copy(w_h, w_v)
        plsc.subcore_barrier()

        wv = [[w_v[r, pl.ds(16 * j, 16)] for j in range(8)] for r in range(5)]
        lanes = lax.iota(jnp.int32, 16)
        bfly = [(lanes ^ k).reshape(16, 1) for k in (1, 2, 4, 8)]
        gdn = lax.GatherDimensionNumbers(
            offset_dims=(), collapsed_slice_dims=(0,), start_index_map=(0,))

        def lane_sum(v):
            for idx in bfly:
                v = v + lax.gather(
                    v, idx, dimension_numbers=gdn, slice_sizes=(1,),
                    mode=lax.GatherScatterMode.PROMISE_IN_BOUNDS)
            return v

        def lane_perm(v, shift):
            idx = ((lanes + shift) & 15).reshape(16, 1)
            return lax.gather(v, idx, dimension_numbers=gdn,
                              slice_sizes=(1,),
                              mode=lax.GatherScatterMode.PROMISE_IN_BOUNDS)

        def fire(m, i, s):
            pltpu.async_copy(xlr_h.at[idx_blk.at[m]], g_v.at[s], sg[s])
            pltpu.async_copy(ea_h.at[pl.ds(base + i * CH, CH)],
                             ea_v.at[s], sea[s])

        def wait_fetch(s):
            pltpu.make_async_copy(xlr_h.at[pl.ds(0, 2 * CH)], g_v.at[s],
                                  sg[s]).wait()
            pltpu.make_async_copy(ea_h.at[pl.ds(0, CH)], ea_v.at[s],
                                  sea[s]).wait()

        def wait_scatter(s):
            pltpu.make_async_copy(acc_o.at[pl.ds(0, CH)], p_v.at[s],
                                  ssc[s]).wait()

        def compute(s):
            gb, eab, pb = g_v.at[s], ea_v.at[s], p_v.at[s]

            def edge(e, c2):
                eav = eab[e, pl.ds(0, 16)]
                a0 = eav[8]
                a1 = eav[9]
                a2 = eav[10]
                a3 = eav[11]
                acc_s0 = jnp.zeros((16,), jnp.float32)
                acc_s1 = jnp.zeros((16,), jnp.float32)
                acc_a0 = jnp.zeros((16,), jnp.float32)
                acc_a1 = jnp.zeros((16,), jnp.float32)
                gl = []
                for j in range(8):
                    g = gb[e, pl.ds(16 * j, 16)]
                    gl.append(g)
                    s0 = g + gb[CH + e, pl.ds(16 * j, 16)]
                    s1 = a0 * wv[0][j] + a1 * wv[1][j]
                    s2 = a2 * wv[2][j] + a3 * wv[3][j]
                    sv = (s0 + s1) + s2
                    if j % 2 == 0:
                        acc_s0 = acc_s0 + wv[4][j] * sv
                        acc_a0 = acc_a0 + wv[4][j] * jnp.abs(sv)
                    else:
                        acc_s1 = acc_s1 + wv[4][j] * sv
                        acc_a1 = acc_a1 + wv[4][j] * jnp.abs(sv)
                exv = jnp.exp(lane_sum(0.6 * (acc_s0 + acc_s1)
                                       + 0.4 * (acc_a0 + acc_a1)))
                for j in range(7):
                    pb[e, pl.ds(16 * j, 16)] = exv * gl[j]
                pg7 = exv * gl[7]
                pb[e, pl.ds(112, 16)] = pg7
                hi = lane_perm(pg7, 8)
                eas = lane_perm(eav, -1)
                spec = jnp.where(lanes == 8, exv, 0.0) \
                    + jnp.where((lanes >= 9) & (lanes <= 13), eas, 0.0)
                pb[e, pl.ds(120, 16)] = jnp.where(lanes < 8, hi, spec)
                return c2
            lax.fori_loop(0, CH, edge, 0)

        def slot_step(kk, b, m, s):
            i = b * NB + m
            wait_fetch(s)

            @pl.when(kk > 0)
            def _():
                wait_scatter(s)
            compute(s)
            pltpu.async_copy(p_v.at[s], acc.at[dst_blk.at[m]], ssc[s],
                             add=True)

            @pl.when(m + 2 < NB)
            def _():
                fire(m + 2, i + 2, s)

        def block(b, carry):
            @pl.when(b > 0)
            def _():
                wait_scatter(0)
                wait_scatter(1)
            pltpu.sync_copy(idx_h.at[wid, pl.ds(b * NB, NB)], idx_blk)
            pltpu.sync_copy(dst_h.at[wid, pl.ds(b * NB, NB)], dst_blk)
            fire(0, b * NB, 0)
            fire(1, b * NB + 1, 1)

            def pair(kk, c2):
                slot_step(kk, b, 2 * kk, 0)
                slot_step(kk, b, 2 * kk + 1, 1)
                return c2
            lax.fori_loop(0, NB // 2, pair, 0)
            return carry
        lax.fori_loop(0, NBLK, block, 0)

        for s in (0, 1):
            wait_scatter(s)

        plsc.subcore_barrier()
        pltpu.sync_copy(acc.at[pl.ds(r0, RPT)],
                        acc_o.at[pl.ds(cid * N + r0, RPT)])

    fn = pl.kernel(body, out_type=out_type, mesh=mesh, scratch_types=scratch,
                   compiler_params=pltpu.CompilerParams(
                       use_tc_tiling_on_sc=False))
    return fn(xlr, idx3, dst3, ea16, w8, z144)[0]


def _k2_body(a0_ref, a1_ref, xl_ref, xr_ref,
             We_ref, att_ref, b_ref, wl2_ref, wr2_ref,
             xl2_ref, xr2_ref, la_ref):
    a = a0_ref[...] + a1_ref[...]
    deg = a[:, D + 5:D + 6]
    la = a[:, D + 1:D + 5] / jnp.maximum(deg, 1.0)
    xl = xl_ref[...]
    s = xl + xr_ref[...] + _mm_t(la, We_ref[...])
    leak = 0.6 * s + 0.4 * jnp.abs(s)
    ex = jnp.exp(jnp.sum(leak * att_ref[...], axis=1, keepdims=True))
    num = a[:, :D] + ex * xl
    den = a[:, D:D + 1] + ex
    h1 = jnp.maximum(num / (den + 1e-16) + b_ref[...], 0.0)
    xl2_ref[...] = _mm_t(h1, wl2_ref[...])
    xr2_ref[...] = _mm_t(h1, wr2_ref[...])
    la_ref[...] = la


def _k2(acc, XL1, XR1, We1, att1, b1, Wl2, Wr2):
    blk = 1000
    row = lambda w: pl.BlockSpec((blk, w), lambda i: (i, 0))
    full = lambda a, b: pl.BlockSpec((a, b), lambda i: (0, 0))
    return pl.pallas_call(
        _k2_body,
        grid=(N // blk,),
        in_specs=[row(PW), row(PW), row(D), row(D),
                  full(D, 4), full(1, D), full(1, D),
                  full(D, D), full(D, D)],
        out_specs=[row(D), row(D), row(4)],
        out_shape=[
            jax.ShapeDtypeStruct((N, D), jnp.float32),
            jax.ShapeDtypeStruct((N, D), jnp.float32),
            jax.ShapeDtypeStruct((N, 4), jnp.float32),
        ],
    )(acc[:N], acc[N:], XL1, XR1,
      We1, att1.reshape(1, D), b1.reshape(1, D), Wl2, Wr2)


def _k3_body(a0_ref, a1_ref, xl_ref, xr_ref, la_ref, We_ref, att_ref,
             b_ref, ids_ref, pipe_ref, gs_ref, Wfc_ref, bfc_ref,
             Wa1_ref, ba1_ref, Wa2_ref, ba2_ref, Wc1_ref, bc1_ref,
             Wc2_ref, bc2_ref, out_ref):
    xl = xl_ref[...]
    s = xl + xr_ref[...] + _mm_t(la_ref[...], We_ref[...])
    leak = 0.6 * s + 0.4 * jnp.abs(s)
    ex = jnp.exp(jnp.sum(leak * att_ref[...], axis=1, keepdims=True))
    a = a0_ref[...] + a1_ref[...]
    num = a[:, :D] + ex * xl
    den = a[:, D:D + 1] + ex
    h = num / (den + 1e-16) + b_ref[...]

    ids = ids_ref[...]
    onehot = (ids == lax.broadcasted_iota(jnp.int32, (1, B), 1)) \
        .astype(jnp.float32)
    cnt = jnp.sum(onehot, axis=0, keepdims=True)
    onehot_m = onehot / jnp.maximum(cnt, 1.0)
    gmean = lax.dot_general(onehot_m, h, (((0,), (0,)), ((), ())),
                            preferred_element_type=jnp.float32)

    niota = lax.broadcasted_iota(jnp.int32, (1, N), 1)
    ohA = (pipe_ref[..., 0:1] == niota).astype(jnp.float32)
    ohB = (pipe_ref[..., 1:2] == niota).astype(jnp.float32)
    pipeA = jnp.dot(ohA, h, preferred_element_type=jnp.float32)
    pipeB = jnp.dot(ohB, h, preferred_element_type=jnp.float32)
    comb = jnp.concatenate([gmean, pipeA, pipeB, gs_ref[...]], axis=1)

    feat = jnp.maximum(_mm_t(comb, Wfc_ref[...]) + bfc_ref[...], 0.0)
    logits = _mm_t(jnp.maximum(_mm_t(feat, Wa1_ref[...]) + ba1_ref[...],
                               0.0), Wa2_ref[...]) + ba2_ref[...]
    vc = jnp.maximum(_mm_t(feat, Wc1_ref[...]) + bc1_ref[...], 0.0)
    value = jnp.sum(vc * Wc2_ref[...], axis=1, keepdims=True) + bc2_ref[...]
    out_ref[...] = jnp.concatenate([logits, value], axis=1)


def _k3(acc, XL2, XR2, la, We2, att2, b2, node_graph_ids,
        pipe_node_indices, global_state, Wfc, bfc, Wa1, ba1, Wa2, ba2,
        Wc1, bc1, Wc2, bc2):
    n_act = Wa2.shape[0]
    vspec = pl.BlockSpec(memory_space=pltpu.VMEM)
    return pl.pallas_call(
        _k3_body,
        in_specs=[vspec] * 21,
        out_specs=vspec,
        out_shape=jax.ShapeDtypeStruct((B, n_act + 1), jnp.float32),
    )(acc[:N], acc[N:], XL2, XR2, la, We2, att2.reshape(1, D),
      b2.reshape(1, D), node_graph_ids.reshape(N, 1), pipe_node_indices,
      global_state, Wfc, bfc.reshape(1, -1), Wa1, ba1.reshape(1, -1),
      Wa2, ba2.reshape(1, -1), Wc1, bc1.reshape(1, -1), Wc2,
      bc2.reshape(1, 1))


def kernel(x, edge_index, edge_attr, node_graph_ids, pipe_node_indices,
           global_state, Wl1, Wr1, We1, att1, b1, Wl2, Wr2, We2, att2, b2,
           Wfc, bfc, Wa1, ba1, Wa2, ba2, Wc1, bc1, Wc2, bc2):
    src = edge_index[0].reshape(NW, NCH, CH)
    dst = edge_index[1].reshape(NW, NCH, CH)
    idx3 = jnp.concatenate([src, dst + N], axis=2)
    XL1, XR1, ea16 = _k1(x, Wl1, Wr1, edge_attr)
    XLR1 = jnp.concatenate([XL1, XR1], axis=0)

    z144 = jnp.zeros((N, PW), jnp.float32)
    w1 = jnp.concatenate([We1.T, att1.reshape(1, D),
                          jnp.zeros((3, D), jnp.float32)], axis=0)
    w2 = jnp.concatenate([We2.T, att2.reshape(1, D),
                          jnp.zeros((3, D), jnp.float32)], axis=0)

    acc1 = _sc_pass(XLR1, idx3, dst, ea16, w1, z144)
    XL2, XR2, la = _k2(acc1, XL1, XR1, We1, att1, b1, Wl2, Wr2)
    XLR2 = jnp.concatenate([XL2, XR2], axis=0)
    acc2 = _sc_pass(XLR2, idx3, dst, ea16, w2, z144)
    return _k3(acc2, XL2, XR2, la, We2, att2, b2, node_graph_ids,
               pipe_node_indices, global_state, Wfc, bfc, Wa1, ba1,
               Wa2, ba2, Wc1, bc1, Wc2, bc2)


# speedup vs baseline: 1.0312x; 1.0312x over previous
import functools

import jax
import jax.numpy as jnp
from jax import lax
from jax.experimental import pallas as pl
from jax.experimental.pallas import tpu as pltpu
from jax.experimental.pallas import tpu_sc as plsc

N = 10000
E = 320000
D = 128
B = 8
NC = 2
NS = 16
NW = NC * NS
EPW = E // NW
CH = 40
NCH = EPW // CH
NB = 50
NBLK = NCH // NB
RPT = N // NS
PW = 136
DW = 16


def _mm_t(a, w):
    return lax.dot_general(a, w, (((1,), (1,)), ((), ())),
                           preferred_element_type=jnp.float32)


def _k1_body(x_ref, wl_ref, wr_ref, ea_ref, xl_ref, xr_ref, ea16_ref):
    xb = x_ref[...]
    xl_ref[...] = _mm_t(xb, wl_ref[...])
    xr_ref[...] = _mm_t(xb, wr_ref[...])
    eblk = ea_ref.shape[0]
    ea16_ref[...] = jnp.concatenate(
        [jnp.zeros((eblk, 8), jnp.float32), ea_ref[...],
         jnp.ones((eblk, 1), jnp.float32),
         jnp.zeros((eblk, 3), jnp.float32)], axis=1)


def _k1(x, Wl, Wr, edge_attr):
    blk = 200
    eblk = E // (N // blk)
    return pl.pallas_call(
        _k1_body,
        grid=(N // blk,),
        in_specs=[
            pl.BlockSpec((blk, D), lambda i: (i, 0)),
            pl.BlockSpec((D, D), lambda i: (0, 0)),
            pl.BlockSpec((D, D), lambda i: (0, 0)),
            pl.BlockSpec((eblk, 4), lambda i: (i, 0)),
        ],
        out_specs=[
            pl.BlockSpec((blk, D), lambda i: (i, 0)),
            pl.BlockSpec((blk, D), lambda i: (i, 0)),
            pl.BlockSpec((eblk, DW), lambda i: (i, 0)),
        ],
        out_shape=[
            jax.ShapeDtypeStruct((N, D), jnp.float32),
            jax.ShapeDtypeStruct((N, D), jnp.float32),
            jax.ShapeDtypeStruct((E, DW), jnp.float32),
        ],
    )(x, Wl, Wr, edge_attr)


def _sc_pass(xlr, idx3, dst3, ea16, w8, z144):
    mesh = plsc.VectorSubcoreMesh(core_axis_name="c", subcore_axis_name="s",
                                  num_cores=NC, num_subcores=NS)

    out_type = [jax.ShapeDtypeStruct((NC * N, PW), jnp.float32)]

    scratch = [
        pltpu.VMEM_SHARED((N, PW), jnp.float32),
        pltpu.VMEM((NB, 2 * CH), jnp.int32),
        pltpu.VMEM((NB, CH), jnp.int32),
        pltpu.VMEM((2, CH, DW), jnp.float32),
        pltpu.VMEM((2, 2 * CH, D), jnp.float32),
        pltpu.VMEM((2, CH, PW), jnp.float32),
        pltpu.VMEM((8, D), jnp.float32),
    ] + [pltpu.SemaphoreType.DMA] * 6

    def body(xlr_h, idx_h, dst_h, ea_h, w_h, z144_h, *rest):
        (acc_o, acc, idx_blk, dst_blk, ea_v, g_v, p_v, w_v,
         *sems) = rest
        sg = sems[0:2]
        sea = sems[2:4]
        ssc = sems[4:6]

        cid = lax.axis_index("c")
        sid = lax.axis_index("s")
        wid = sid * NC + cid
        base = wid * EPW
        r0 = sid * RPT

        pltpu.sync_copy(z144_h.at[pl.ds(r0, RPT)], acc.at[pl.ds(r0, RPT)])
        pltpu.sync_copy(w_h, w_v)
        plsc.subcore_barrier()

        wv = [[w_v[r, pl.ds(16 * j, 16)] for j in range(8)] for r in range(5)]
        lanes = lax.iota(jnp.int32, 16)
        bfly = [(lanes ^ k).reshape(16, 1) for k in (1, 2, 4, 8)]
        gdn = lax.GatherDimensionNumbers(
            offset_dims=(), collapsed_slice_dims=(0,), start_index_map=(0,))

        def lane_sum(v):
            for idx in bfly:
                v = v + lax.gather(
                    v, idx, dimension_numbers=gdn, slice_sizes=(1,),
                    mode=lax.GatherScatterMode.PROMISE_IN_BOUNDS)
            return v

        def lane_perm(v, shift):
            idx = ((lanes + shift) & 15).reshape(16, 1)
            return lax.gather(v, idx, dimension_numbers=gdn,
                              slice_sizes=(1,),
                              mode=lax.GatherScatterMode.PROMISE_IN_BOUNDS)

        def fire(m, i, s):
            pltpu.async_copy(xlr_h.at[idx_blk.at[m]], g_v.at[s], sg[s])
            pltpu.async_copy(ea_h.at[pl.ds(base + i * CH, CH)],
                             ea_v.at[s], sea[s])

        def wait_fetch(s):
            pltpu.make_async_copy(xlr_h.at[pl.ds(0, 2 * CH)], g_v.at[s],
                                  sg[s]).wait()
            pltpu.make_async_copy(ea_h.at[pl.ds(0, CH)], ea_v.at[s],
                                  sea[s]).wait()

        def wait_scatter(s):
            pltpu.make_async_copy(acc_o.at[pl.ds(0, CH)], p_v.at[s],
                                  ssc[s]).wait()

        def compute(s):
            gb, eab, pb = g_v.at[s], ea_v.at[s], p_v.at[s]

            @plsc.parallel_loop(0, CH, 1, unroll=2)
            def edge(e):
                eav = eab[e, pl.ds(0, 16)]
                a0 = eav[8]
                a1 = eav[9]
                a2 = eav[10]
                a3 = eav[11]
                acc_s0 = jnp.zeros((16,), jnp.float32)
                acc_s1 = jnp.zeros((16,), jnp.float32)
                acc_a0 = jnp.zeros((16,), jnp.float32)
                acc_a1 = jnp.zeros((16,), jnp.float32)
                gl = []
                for j in range(8):
                    g = gb[e, pl.ds(16 * j, 16)]
                    gl.append(g)
                    s0 = g + gb[CH + e, pl.ds(16 * j, 16)]
                    s1 = a0 * wv[0][j] + a1 * wv[1][j]
                    s2 = a2 * wv[2][j] + a3 * wv[3][j]
                    sv = (s0 + s1) + s2
                    if j % 2 == 0:
                        acc_s0 = acc_s0 + wv[4][j] * sv
                        acc_a0 = acc_a0 + wv[4][j] * jnp.abs(sv)
                    else:
                        acc_s1 = acc_s1 + wv[4][j] * sv
                        acc_a1 = acc_a1 + wv[4][j] * jnp.abs(sv)
                exv = jnp.exp(lane_sum(0.6 * (acc_s0 + acc_s1)
                                       + 0.4 * (acc_a0 + acc_a1)))
                for j in range(7):
                    pb[e, pl.ds(16 * j, 16)] = exv * gl[j]
                pg7 = exv * gl[7]
                pb[e, pl.ds(112, 16)] = pg7
                hi = lane_perm(pg7, 8)
                eas = lane_perm(eav, -1)
                spec = jnp.where(lanes == 8, exv, 0.0) \
                    + jnp.where((lanes >= 9) & (lanes <= 13), eas, 0.0)
                pb[e, pl.ds(120, 16)] = jnp.where(lanes < 8, hi, spec)

        def slot_step(kk, b, m, s):
            i = b * NB + m
            wait_fetch(s)

            @pl.when(kk > 0)
            def _():
                wait_scatter(s)
            compute(s)
            pltpu.async_copy(p_v.at[s], acc.at[dst_blk.at[m]], ssc[s],
                             add=True)

            @pl.when(m + 2 < NB)
            def _():
                fire(m + 2, i + 2, s)

        def block(b, carry):
            @pl.when(b > 0)
            def _():
                wait_scatter(0)
                wait_scatter(1)
            pltpu.sync_copy(idx_h.at[wid, pl.ds(b * NB, NB)], idx_blk)
            pltpu.sync_copy(dst_h.at[wid, pl.ds(b * NB, NB)], dst_blk)
            fire(0, b * NB, 0)
            fire(1, b * NB + 1, 1)

            def pair(kk, c2):
                slot_step(kk, b, 2 * kk, 0)
                slot_step(kk, b, 2 * kk + 1, 1)
                return c2
            lax.fori_loop(0, NB // 2, pair, 0)
            return carry
        lax.fori_loop(0, NBLK, block, 0)

        for s in (0, 1):
            wait_scatter(s)

        plsc.subcore_barrier()
        pltpu.sync_copy(acc.at[pl.ds(r0, RPT)],
                        acc_o.at[pl.ds(cid * N + r0, RPT)])

    fn = pl.kernel(body, out_type=out_type, mesh=mesh, scratch_types=scratch,
                   compiler_params=pltpu.CompilerParams(
                       use_tc_tiling_on_sc=False))
    return fn(xlr, idx3, dst3, ea16, w8, z144)[0]


def _k2_body(a0_ref, a1_ref, xl_ref, xr_ref,
             We_ref, att_ref, b_ref, wl2_ref, wr2_ref,
             xl2_ref, xr2_ref, la_ref):
    a = a0_ref[...] + a1_ref[...]
    deg = a[:, D + 5:D + 6]
    la = a[:, D + 1:D + 5] / jnp.maximum(deg, 1.0)
    xl = xl_ref[...]
    s = xl + xr_ref[...] + _mm_t(la, We_ref[...])
    leak = 0.6 * s + 0.4 * jnp.abs(s)
    ex = jnp.exp(jnp.sum(leak * att_ref[...], axis=1, keepdims=True))
    num = a[:, :D] + ex * xl
    den = a[:, D:D + 1] + ex
    h1 = jnp.maximum(num / (den + 1e-16) + b_ref[...], 0.0)
    xl2_ref[...] = _mm_t(h1, wl2_ref[...])
    xr2_ref[...] = _mm_t(h1, wr2_ref[...])
    la_ref[...] = la


def _k2(acc, XL1, XR1, We1, att1, b1, Wl2, Wr2):
    blk = 1000
    row = lambda w: pl.BlockSpec((blk, w), lambda i: (i, 0))
    full = lambda a, b: pl.BlockSpec((a, b), lambda i: (0, 0))
    return pl.pallas_call(
        _k2_body,
        grid=(N // blk,),
        in_specs=[row(PW), row(PW), row(D), row(D),
                  full(D, 4), full(1, D), full(1, D),
                  full(D, D), full(D, D)],
        out_specs=[row(D), row(D), row(4)],
        out_shape=[
            jax.ShapeDtypeStruct((N, D), jnp.float32),
            jax.ShapeDtypeStruct((N, D), jnp.float32),
            jax.ShapeDtypeStruct((N, 4), jnp.float32),
        ],
    )(acc[:N], acc[N:], XL1, XR1,
      We1, att1.reshape(1, D), b1.reshape(1, D), Wl2, Wr2)


def _k3_body(a0_ref, a1_ref, xl_ref, xr_ref, la_ref, We_ref, att_ref,
             b_ref, ids_ref, pipe_ref, gs_ref, Wfc_ref, bfc_ref,
             Wa1_ref, ba1_ref, Wa2_ref, ba2_ref, Wc1_ref, bc1_ref,
             Wc2_ref, bc2_ref, out_ref):
    xl = xl_ref[...]
    s = xl + xr_ref[...] + _mm_t(la_ref[...], We_ref[...])
    leak = 0.6 * s + 0.4 * jnp.abs(s)
    ex = jnp.exp(jnp.sum(leak * att_ref[...], axis=1, keepdims=True))
    a = a0_ref[...] + a1_ref[...]
    num = a[:, :D] + ex * xl
    den = a[:, D:D + 1] + ex
    h = num / (den + 1e-16) + b_ref[...]

    ids = ids_ref[...]
    onehot = (ids == lax.broadcasted_iota(jnp.int32, (1, B), 1)) \
        .astype(jnp.float32)
    cnt = jnp.sum(onehot, axis=0, keepdims=True)
    onehot_m = onehot / jnp.maximum(cnt, 1.0)
    gmean = lax.dot_general(onehot_m, h, (((0,), (0,)), ((), ())),
                            preferred_element_type=jnp.float32)

    niota = lax.broadcasted_iota(jnp.int32, (1, N), 1)
    ohA = (pipe_ref[..., 0:1] == niota).astype(jnp.float32)
    ohB = (pipe_ref[..., 1:2] == niota).astype(jnp.float32)
    pipeA = jnp.dot(ohA, h, preferred_element_type=jnp.float32)
    pipeB = jnp.dot(ohB, h, preferred_element_type=jnp.float32)
    comb = jnp.concatenate([gmean, pipeA, pipeB, gs_ref[...]], axis=1)

    feat = jnp.maximum(_mm_t(comb, Wfc_ref[...]) + bfc_ref[...], 0.0)
    logits = _mm_t(jnp.maximum(_mm_t(feat, Wa1_ref[...]) + ba1_ref[...],
                               0.0), Wa2_ref[...]) + ba2_ref[...]
    vc = jnp.maximum(_mm_t(feat, Wc1_ref[...]) + bc1_ref[...], 0.0)
    value = jnp.sum(vc * Wc2_ref[...], axis=1, keepdims=True) + bc2_ref[...]
    out_ref[...] = jnp.concatenate([logits, value], axis=1)


def _k3(acc, XL2, XR2, la, We2, att2, b2, node_graph_ids,
        pipe_node_indices, global_state, Wfc, bfc, Wa1, ba1, Wa2, ba2,
        Wc1, bc1, Wc2, bc2):
    n_act = Wa2.shape[0]
    vspec = pl.BlockSpec(memory_space=pltpu.VMEM)
    return pl.pallas_call(
        _k3_body,
        in_specs=[vspec] * 21,
        out_specs=vspec,
        out_shape=jax.ShapeDtypeStruct((B, n_act + 1), jnp.float32),
    )(acc[:N], acc[N:], XL2, XR2, la, We2, att2.reshape(1, D),
      b2.reshape(1, D), node_graph_ids.reshape(N, 1), pipe_node_indices,
      global_state, Wfc, bfc.reshape(1, -1), Wa1, ba1.reshape(1, -1),
      Wa2, ba2.reshape(1, -1), Wc1, bc1.reshape(1, -1), Wc2,
      bc2.reshape(1, 1))


def kernel(x, edge_index, edge_attr, node_graph_ids, pipe_node_indices,
           global_state, Wl1, Wr1, We1, att1, b1, Wl2, Wr2, We2, att2, b2,
           Wfc, bfc, Wa1, ba1, Wa2, ba2, Wc1, bc1, Wc2, bc2):
    src = edge_index[0].reshape(NW, NCH, CH)
    dst = edge_index[1].reshape(NW, NCH, CH)
    idx3 = jnp.concatenate([src, dst + N], axis=2)
    XL1, XR1, ea16 = _k1(x, Wl1, Wr1, edge_attr)
    XLR1 = jnp.concatenate([XL1, XR1], axis=0)

    z144 = jnp.zeros((N, PW), jnp.float32)
    w1 = jnp.concatenate([We1.T, att1.reshape(1, D),
                          jnp.zeros((3, D), jnp.float32)], axis=0)
    w2 = jnp.concatenate([We2.T, att2.reshape(1, D),
                          jnp.zeros((3, D), jnp.float32)], axis=0)

    acc1 = _sc_pass(XLR1, idx3, dst, ea16, w1, z144)
    XL2, XR2, la = _k2(acc1, XL1, XR1, We1, att1, b1, Wl2, Wr2)
    XLR2 = jnp.concatenate([XL2, XR2], axis=0)
    acc2 = _sc_pass(XLR2, idx3, dst, ea16, w2, z144)
    return _k3(acc2, XL2, XR2, la, We2, att2, b2, node_graph_ids,
               pipe_node_indices, global_state, Wfc, bfc, Wa1, ba1,
               Wa2, ba2, Wc1, bc1, Wc2, bc2)
